# baseline (device time: 23159 ns/iter reference)
import functools

import jax
import jax.numpy as jnp
from jax import lax
from jax.experimental import pallas as pl
from jax.experimental.pallas import tpu as pltpu

import os as _os
try:
    with open(_os.path.join(_os.path.dirname(_os.path.abspath(__file__)),
                            "ablate.txt")) as _f:
        _ABLATE = _f.read().strip()
except OSError:
    _ABLATE = ""

N_DEV = 8
XOR_MASKS = (1, 3, 4)
PARTS = ((0, 96), (96, 80), (176, 80), (256, 96), (352, 80), (432, 80))
ORDERS = ((0, 1, 2), (1, 2, 0), (2, 0, 1)) * 2
B, SQ, D_MODEL = 2, 256, 512
H_LOCAL, DH = 4, 64
D_HEADS = H_LOCAL * DH


def kernel(x, Wq, K_ext, V_ext, Wo):
    my = lax.axis_index("i")
    wo = lax.dynamic_slice_in_dim(Wo, my * D_HEADS, D_HEADS, axis=0)
    wo = wo.astype(jnp.bfloat16)
    k = K_ext.reshape(B, SQ, D_HEADS)
    v = V_ext.reshape(B, SQ, D_HEADS)

    return pl.pallas_call(
        _body,
        out_shape=jax.ShapeDtypeStruct((B, SQ, D_MODEL), jnp.bfloat16),
        in_specs=[
            pl.BlockSpec(memory_space=pltpu.VMEM),
            pl.BlockSpec(memory_space=pl.ANY),
            pl.BlockSpec(memory_space=pltpu.VMEM),
            pl.BlockSpec(memory_space=pltpu.VMEM),
            pl.BlockSpec(memory_space=pltpu.VMEM),
        ],
        out_specs=pl.BlockSpec(memory_space=pltpu.VMEM),
        scratch_shapes=[
            pltpu.VMEM((B * SQ, D_HEADS), jnp.bfloat16),
            pltpu.VMEM((3, B * SQ, D_MODEL), jnp.bfloat16),
            pltpu.VMEM((3, B * SQ, D_MODEL), jnp.bfloat16),
            pltpu.VMEM((D_MODEL, D_HEADS), jnp.float32),
            pltpu.SemaphoreType.DMA((6, 3)),
            pltpu.SemaphoreType.DMA((6, 3)),
            pltpu.SemaphoreType.DMA,
        ],
        compiler_params=(pltpu.CompilerParams() if _ABLATE == "compute_only"
                         else pltpu.CompilerParams(collective_id=0)),
    )(x, Wq, k, v, wo)


def _body(x_ref, wq_ref, k_ref, v_ref, wo_ref, out_ref, ctx_ref, send_ref,
          recv_ref, wq_vmem, send_sems, recv_sems, wq_sem):
    my = lax.axis_index("i")

    wq_cp = pltpu.make_async_copy(
        wq_ref.at[:, pl.ds(my * D_HEADS, D_HEADS)], wq_vmem, wq_sem)
    wq_cp.start()

    qb = lax.broadcasted_iota(jnp.int32, (SQ, SQ), 0) // 64
    kb = lax.broadcasted_iota(jnp.int32, (SQ, SQ), 1) // 64
    mask = (qb == kb) | (kb == 0) | ((qb + kb) % 3 == 0)

    wq_cp.wait()
    wq_bf = wq_vmem[...].astype(jnp.bfloat16)

    def compute_batch(b):
        xb = x_ref[b].astype(jnp.bfloat16)
        q_b = jnp.dot(xb, wq_bf, preferred_element_type=jnp.float32)
        q_b = q_b.astype(jnp.bfloat16)
        for h in range(H_LOCAL):
            qh = q_b[:, h * DH:(h + 1) * DH]
            kh = k_ref[b][:, h * DH:(h + 1) * DH].astype(jnp.bfloat16)
            s = lax.dot_general(qh, kh, (((1,), (1,)), ((), ())),
                                preferred_element_type=jnp.float32) * 0.125
            e = jnp.exp(jnp.where(mask, s, -1e9))
            w = (e / jnp.sum(e, axis=1, keepdims=True)).astype(jnp.bfloat16)
            vh = v_ref[b][:, h * DH:(h + 1) * DH].astype(jnp.bfloat16)
            ctx = jnp.dot(w, vh, preferred_element_type=jnp.float32)
            ctx_ref[b * SQ:(b + 1) * SQ, h * DH:(h + 1) * DH] = (
                ctx.astype(jnp.bfloat16))
        send_ref[0, b * SQ:(b + 1) * SQ, :] = jnp.dot(
            ctx_ref[b * SQ:(b + 1) * SQ, :], wo_ref[...],
            preferred_element_type=jnp.float32).astype(jnp.bfloat16)

    def _rdma(p, r):
        off, n = PARTS[p]
        sl = pl.ds(off, n)
        return pltpu.make_async_remote_copy(
            src_ref=send_ref.at[r, sl, :],
            dst_ref=recv_ref.at[r, sl, :],
            send_sem=send_sems.at[p, r],
            recv_sem=recv_sems.at[p, r],
            device_id=(my ^ XOR_MASKS[ORDERS[p][r]],),
            device_id_type=pl.DeviceIdType.MESH,
        )

    rdmas = {(p, r): _rdma(p, r) for p in range(len(PARTS)) for r in range(3)}

    if _ABLATE == "comm_only":
        send_ref[0] = jnp.zeros((B * SQ, D_MODEL), jnp.bfloat16)
    else:
        compute_batch(0)
    if _ABLATE == "compute_only":
        compute_batch(1)
        out_ref[0] = send_ref[0, :SQ, :]
        out_ref[1] = send_ref[0, SQ:, :]
        return

    barrier = pltpu.get_barrier_semaphore()
    for m in XOR_MASKS:
        pl.semaphore_signal(barrier, inc=1, device_id=(my ^ m,),
                            device_id_type=pl.DeviceIdType.MESH)
    pl.semaphore_wait(barrier, len(XOR_MASKS))

    for p in range(3):
        rdmas[p, 0].start()
    if _ABLATE != "comm_only":
        compute_batch(1)
    for p in range(3, len(PARTS)):
        rdmas[p, 0].start()

    for r in range(3):
        for p in range(len(PARTS)):
            off, n = PARTS[p]
            sl = slice(off, off + n)
            rdmas[p, r].wait()
            if r < 2:
                send_ref[r + 1, sl, :] = (
                    send_ref[r, sl, :] + recv_ref[r, sl, :])
                rdmas[p, r + 1].start()
            else:
                bp, off_in = off // SQ, off % SQ
                out_ref[bp, off_in:off_in + n, :] = (
                    send_ref[r, sl, :] + recv_ref[r, sl, :])

    @functools.partial(pl.run_scoped, exit_sem=pltpu.SemaphoreType.REGULAR)
    def _(exit_sem):
        for m in XOR_MASKS:
            pl.semaphore_signal(exit_sem, inc=1, device_id=(my ^ m,),
                                device_id_type=pl.DeviceIdType.MESH)
        pl.semaphore_wait(exit_sem, len(XOR_MASKS))


# device time: 20397 ns/iter; 1.1354x vs baseline; 1.1354x over previous
import functools

import jax
import jax.numpy as jnp
from jax import lax
from jax.experimental import pallas as pl
from jax.experimental.pallas import tpu as pltpu

import os as _os
try:
    with open(_os.path.join(_os.path.dirname(_os.path.abspath(__file__)),
                            "ablate.txt")) as _f:
        _ABLATE = _f.read().strip()
except OSError:
    _ABLATE = ""

N_DEV = 8
XOR_MASKS = (1, 3, 4)
PARTS = ((0, 48), (48, 40), (88, 40), (128, 48), (168, 40), (208, 48),
         (256, 48), (304, 40), (344, 40), (384, 48), (424, 40), (464, 48))
ORDERS = ((0, 1, 2), (1, 2, 0), (2, 0, 1)) * 4
B, SQ, D_MODEL = 2, 256, 512
H_LOCAL, DH = 4, 64
D_HEADS = H_LOCAL * DH


def kernel(x, Wq, K_ext, V_ext, Wo):
    my = lax.axis_index("i")
    wo = lax.dynamic_slice_in_dim(Wo, my * D_HEADS, D_HEADS, axis=0)
    wo = wo.astype(jnp.bfloat16)
    x3 = x.astype(jnp.bfloat16)
    k = K_ext.reshape(B, SQ, D_HEADS).astype(jnp.bfloat16)
    v = V_ext.reshape(B, SQ, D_HEADS).astype(jnp.bfloat16)

    return pl.pallas_call(
        _body,
        out_shape=jax.ShapeDtypeStruct((B, SQ, D_MODEL), jnp.bfloat16),
        in_specs=[
            pl.BlockSpec(memory_space=pltpu.VMEM),
            pl.BlockSpec(memory_space=pl.ANY),
            pl.BlockSpec(memory_space=pltpu.VMEM),
            pl.BlockSpec(memory_space=pltpu.VMEM),
            pl.BlockSpec(memory_space=pltpu.VMEM),
        ],
        out_specs=pl.BlockSpec(memory_space=pltpu.VMEM),
        scratch_shapes=[
            pltpu.VMEM((B * SQ, D_HEADS), jnp.bfloat16),
            pltpu.VMEM((3, B * SQ, D_MODEL), jnp.bfloat16),
            pltpu.VMEM((3, B * SQ, D_MODEL), jnp.bfloat16),
            pltpu.VMEM((D_MODEL, D_HEADS), jnp.float32),
            pltpu.SemaphoreType.DMA((12, 3)),
            pltpu.SemaphoreType.DMA((12, 3)),
            pltpu.SemaphoreType.DMA,
        ],
        compiler_params=(pltpu.CompilerParams() if _ABLATE == "compute_only"
                         else pltpu.CompilerParams(collective_id=0)),
    )(x3, Wq, k, v, wo)


def _body(x_ref, wq_ref, k_ref, v_ref, wo_ref, out_ref, ctx_ref, send_ref,
          recv_ref, wq_vmem, send_sems, recv_sems, wq_sem):
    my = lax.axis_index("i")

    wq_cp = pltpu.make_async_copy(
        wq_ref.at[:, pl.ds(my * D_HEADS, D_HEADS)], wq_vmem, wq_sem)
    wq_cp.start()

    qb = lax.broadcasted_iota(jnp.int32, (SQ, SQ), 0) // 64
    kb = lax.broadcasted_iota(jnp.int32, (SQ, SQ), 1) // 64
    mask = (qb == kb) | (kb == 0) | ((qb + kb) % 3 == 0)

    wq_cp.wait()
    wq_bf = wq_vmem[...].astype(jnp.bfloat16)

    def compute_batch(b):
        xb = x_ref[b]
        q_b = jnp.dot(xb, wq_bf, preferred_element_type=jnp.float32)
        q_b = q_b.astype(jnp.bfloat16)
        for h in range(H_LOCAL):
            qh = q_b[:, h * DH:(h + 1) * DH]
            kh = k_ref[b][:, h * DH:(h + 1) * DH]
            s = lax.dot_general(qh, kh, (((1,), (1,)), ((), ())),
                                preferred_element_type=jnp.float32) * 0.125
            e = jnp.exp(jnp.where(mask, s, -1e9))
            w = (e / jnp.sum(e, axis=1, keepdims=True)).astype(jnp.bfloat16)
            ctx = jnp.dot(w, v_ref[b][:, h * DH:(h + 1) * DH],
                          preferred_element_type=jnp.float32)
            ctx_ref[b * SQ:(b + 1) * SQ, h * DH:(h + 1) * DH] = (
                ctx.astype(jnp.bfloat16))
        send_ref[0, b * SQ:(b + 1) * SQ, :] = jnp.dot(
            ctx_ref[b * SQ:(b + 1) * SQ, :], wo_ref[...],
            preferred_element_type=jnp.float32).astype(jnp.bfloat16)

    def _rdma(p, r):
        off, n = PARTS[p]
        sl = pl.ds(off, n)
        return pltpu.make_async_remote_copy(
            src_ref=send_ref.at[r, sl, :],
            dst_ref=recv_ref.at[r, sl, :],
            send_sem=send_sems.at[p, r],
            recv_sem=recv_sems.at[p, r],
            device_id=(my ^ XOR_MASKS[ORDERS[p][r]],),
            device_id_type=pl.DeviceIdType.MESH,
        )

    rdmas = {(p, r): _rdma(p, r) for p in range(len(PARTS)) for r in range(3)}

    if _ABLATE == "comm_only":
        send_ref[0] = jnp.zeros((B * SQ, D_MODEL), jnp.bfloat16)
    else:
        compute_batch(0)
    if _ABLATE == "compute_only":
        compute_batch(1)
        out_ref[0] = send_ref[0, :SQ, :]
        out_ref[1] = send_ref[0, SQ:, :]
        return

    barrier = pltpu.get_barrier_semaphore()
    for m in XOR_MASKS:
        pl.semaphore_signal(barrier, inc=1, device_id=(my ^ m,),
                            device_id_type=pl.DeviceIdType.MESH)
    pl.semaphore_wait(barrier, len(XOR_MASKS))

    nhalf = len(PARTS) // 2
    for p in range(nhalf):
        rdmas[p, 0].start()
    if _ABLATE != "comm_only":
        compute_batch(1)
    for p in range(nhalf, len(PARTS)):
        rdmas[p, 0].start()

    for r in range(3):
        for p in range(len(PARTS)):
            off, n = PARTS[p]
            sl = slice(off, off + n)
            rdmas[p, r].wait()
            if r < 2:
                send_ref[r + 1, sl, :] = (
                    send_ref[r, sl, :] + recv_ref[r, sl, :])
                rdmas[p, r + 1].start()
            else:
                bp, off_in = off // SQ, off % SQ
                out_ref[bp, off_in:off_in + n, :] = (
                    send_ref[r, sl, :] + recv_ref[r, sl, :])

    @functools.partial(pl.run_scoped, exit_sem=pltpu.SemaphoreType.REGULAR)
    def _(exit_sem):
        for m in XOR_MASKS:
            pl.semaphore_signal(exit_sem, inc=1, device_id=(my ^ m,),
                                device_id_type=pl.DeviceIdType.MESH)
        pl.semaphore_wait(exit_sem, len(XOR_MASKS))
